# baseline (speedup 1.0000x reference)
import jax
import jax.numpy as jnp
from jax.experimental import pallas as pl
from jax.experimental.pallas import tpu as pltpu

_BLK = 4096
_NBUF = 4


def _make_kernel(nblk, tail):
    def body(x_ref, w_ref, b_ref, o_hbm, obuf, tailbuf, sem, tsem):
        i = pl.program_id(0)
        slot = jax.lax.rem(i, _NBUF)

        @pl.when(i >= _NBUF)
        def _():
            pltpu.make_async_copy(
                obuf.at[slot],
                o_hbm.at[:, pl.ds((i - _NBUF) * _BLK, _BLK)],
                sem.at[slot],
            ).wait()

        blk = jax.lax.dot_general(
            x_ref[...], w_ref[...],
            (((1,), (1,)), ((), ())),
            preferred_element_type=jnp.float32,
        ) + b_ref[...]

        @pl.when(i < nblk - 1)
        def _():
            obuf[slot] = blk
            pltpu.make_async_copy(
                obuf.at[slot],
                o_hbm.at[:, pl.ds(i * _BLK, _BLK)],
                sem.at[slot],
            ).start()

        @pl.when(i == nblk - 1)
        def _():
            tailbuf[...] = blk[:, 0:tail]
            tail_cp = pltpu.make_async_copy(
                tailbuf,
                o_hbm.at[:, pl.ds((nblk - 1) * _BLK, tail)],
                tsem,
            )
            tail_cp.start()
            for j in range(nblk - _NBUF, nblk - 1):
                s = j % _NBUF
                pltpu.make_async_copy(
                    obuf.at[s],
                    o_hbm.at[:, pl.ds(j * _BLK, _BLK)],
                    sem.at[s],
                ).wait()
            tail_cp.wait()

    return body


def kernel(x, y, freeze_flag, W, b):
    del y, freeze_flag
    Bm, D = x.shape
    C1 = W.shape[0]
    nblk = pl.cdiv(C1, _BLK)
    tail = C1 - (nblk - 1) * _BLK
    b2 = b.reshape(1, C1)
    out = pl.pallas_call(
        _make_kernel(nblk, tail),
        grid=(nblk,),
        in_specs=[
            pl.BlockSpec((Bm, D), lambda i: (0, 0)),
            pl.BlockSpec((_BLK, D), lambda i: (i, 0)),
            pl.BlockSpec((1, _BLK), lambda i: (0, i)),
        ],
        out_specs=pl.BlockSpec(memory_space=pl.ANY),
        out_shape=jax.ShapeDtypeStruct((Bm, C1), jnp.float32),
        scratch_shapes=[
            pltpu.VMEM((_NBUF, Bm, _BLK), jnp.float32),
            pltpu.VMEM((Bm, tail), jnp.float32),
            pltpu.SemaphoreType.DMA((_NBUF,)),
            pltpu.SemaphoreType.DMA,
        ],
        compiler_params=pltpu.CompilerParams(
            dimension_semantics=("arbitrary",),
        ),
    )(x, W, b2)
    return out


# speedup vs baseline: 1.0667x; 1.0667x over previous
import jax
import jax.numpy as jnp
from jax.experimental import pallas as pl
from jax.experimental.pallas import tpu as pltpu

_BLK = 4096
_NBUF = 4


def _make_kernel(nblk, tail):
    def body(x_ref, b_ref, w_hbm, o_hbm,
             wbuf, wtailbuf, obuf, tailbuf, wsem, wtsem, osem, otsem):
        i = pl.program_id(0)
        slot = jax.lax.rem(i, _NBUF)

        @pl.when(i == 0)
        def _():
            for k in range(_NBUF):
                pltpu.make_async_copy(
                    w_hbm.at[pl.ds(k * _BLK, _BLK), :],
                    wbuf.at[k],
                    wsem.at[k],
                ).start()

        @pl.when(i < nblk - 1)
        def _():
            pltpu.make_async_copy(
                w_hbm.at[pl.ds(i * _BLK, _BLK), :],
                wbuf.at[slot],
                wsem.at[slot],
            ).wait()

        @pl.when(i == nblk - 1)
        def _():
            pltpu.make_async_copy(
                w_hbm.at[pl.ds((nblk - 1) * _BLK, tail), :],
                wtailbuf,
                wtsem,
            ).wait()

        @pl.when(i >= _NBUF)
        def _():
            pltpu.make_async_copy(
                obuf.at[slot],
                o_hbm.at[:, pl.ds((i - _NBUF) * _BLK, _BLK)],
                osem.at[slot],
            ).wait()

        @pl.when(i < nblk - 1)
        def _():
            obuf[slot] = jax.lax.dot_general(
                x_ref[...], wbuf[slot],
                (((1,), (1,)), ((), ())),
                preferred_element_type=jnp.float32,
            ) + b_ref[...]
            pltpu.make_async_copy(
                obuf.at[slot],
                o_hbm.at[:, pl.ds(i * _BLK, _BLK)],
                osem.at[slot],
            ).start()
            j = i + _NBUF

            @pl.when(j < nblk - 1)
            def _():
                pltpu.make_async_copy(
                    w_hbm.at[pl.ds(j * _BLK, _BLK), :],
                    wbuf.at[slot],
                    wsem.at[slot],
                ).start()

            @pl.when(j == nblk - 1)
            def _():
                pltpu.make_async_copy(
                    w_hbm.at[pl.ds((nblk - 1) * _BLK, tail), :],
                    wtailbuf,
                    wtsem,
                ).start()

        @pl.when(i == nblk - 1)
        def _():
            tailbuf[...] = jax.lax.dot_general(
                x_ref[...], wtailbuf[...],
                (((1,), (1,)), ((), ())),
                preferred_element_type=jnp.float32,
            ) + b_ref[:, 0:tail]
            tail_cp = pltpu.make_async_copy(
                tailbuf,
                o_hbm.at[:, pl.ds((nblk - 1) * _BLK, tail)],
                otsem,
            )
            tail_cp.start()
            for j in range(nblk - _NBUF, nblk - 1):
                s = j % _NBUF
                pltpu.make_async_copy(
                    obuf.at[s],
                    o_hbm.at[:, pl.ds(j * _BLK, _BLK)],
                    osem.at[s],
                ).wait()
            tail_cp.wait()

    return body


def kernel(x, y, freeze_flag, W, b):
    del y, freeze_flag
    Bm, D = x.shape
    C1 = W.shape[0]
    nblk = pl.cdiv(C1, _BLK)
    tail = C1 - (nblk - 1) * _BLK
    b2 = b.reshape(1, C1)
    out = pl.pallas_call(
        _make_kernel(nblk, tail),
        grid=(nblk,),
        in_specs=[
            pl.BlockSpec((Bm, D), lambda i: (0, 0)),
            pl.BlockSpec((1, _BLK), lambda i: (0, i)),
            pl.BlockSpec(memory_space=pl.ANY),
        ],
        out_specs=pl.BlockSpec(memory_space=pl.ANY),
        out_shape=jax.ShapeDtypeStruct((Bm, C1), jnp.float32),
        scratch_shapes=[
            pltpu.VMEM((_NBUF, _BLK, D), jnp.float32),
            pltpu.VMEM((tail, D), jnp.float32),
            pltpu.VMEM((_NBUF, Bm, _BLK), jnp.float32),
            pltpu.VMEM((Bm, tail), jnp.float32),
            pltpu.SemaphoreType.DMA((_NBUF,)),
            pltpu.SemaphoreType.DMA,
            pltpu.SemaphoreType.DMA((_NBUF,)),
            pltpu.SemaphoreType.DMA,
        ],
        compiler_params=pltpu.CompilerParams(
            dimension_semantics=("arbitrary",),
        ),
    )(x, b2, W)
    return out
